# Initial kernel scaffold; baseline (speedup 1.0000x reference)
import jax
import jax.numpy as jnp
from jax.experimental import pallas as pl


def kernel(x1, x2, w_1x1, b_1x1, w_conv_a, b_conv_a, bn_a_gamma, bn_a_beta, bn_a_mean, bn_a_var, w_conv_b, b_conv_b, bn_b_gamma, bn_b_beta, bn_b_mean, bn_b_var):
    raise NotImplementedError("write your pallas kernel here")


# speedup vs baseline: 1.2810x; 1.2810x over previous
import jax
import jax.numpy as jnp
from jax import lax
from jax.experimental import pallas as pl
from jax.experimental.pallas import tpu as pltpu

BN_EPS = 1e-5
SLOPE = 0.01


def _interp_matrix(n_in):
    n_out = 2 * n_in
    s = jnp.arange(n_out, dtype=jnp.float32) * (n_in - 1) / (n_out - 1)
    i0 = jnp.floor(s).astype(jnp.int32)
    i1 = jnp.minimum(i0 + 1, n_in - 1)
    frac = s - i0.astype(jnp.float32)
    m = jnp.zeros((n_out, n_in), jnp.float32)
    m = m.at[jnp.arange(n_out), i0].add(1.0 - frac)
    m = m.at[jnp.arange(n_out), i1].add(frac)
    return m


def _conv1x1_kernel(x_ref, w_ref, b_ref, o_ref):
    x = x_ref[0].astype(jnp.bfloat16)
    y = jnp.dot(w_ref[...], x, preferred_element_type=jnp.float32)
    o_ref[0] = y + b_ref[...]


def _make_upsample_kernel(c2, h, w):
    h2, w2 = 2 * h, 2 * w

    def body(x_ref, uyb_ref, uxt_ref, o_ref):
        t = jnp.dot(x_ref[0], uxt_ref[...],
                    preferred_element_type=jnp.float32)
        t = t.reshape(c2, h, w2)
        o = lax.dot_general(uyb_ref[...], t, (((2,), (1,)), ((0,), (0,))),
                            preferred_element_type=jnp.float32)
        o_ref[0] = o.astype(jnp.bfloat16)

    return body


def _make_conv_kernel(n_in, h, w, pad_out):
    pitch = w + 2
    n_rows = h * pitch

    def body(*refs):
        x_refs = refs[:n_in]
        w_refs = refs[n_in:2 * n_in]
        scale_ref, shift_ref, o_ref = refs[2 * n_in:]
        c_out = o_ref.shape[-1]
        acc = jnp.zeros((n_rows, c_out), jnp.float32)
        for x_ref, w_ref in zip(x_refs, w_refs):
            xin = x_ref[0]
            for dy in range(3):
                for dx in range(3):
                    off = dy * pitch + dx
                    acc = acc + jnp.dot(xin[off:off + n_rows, :],
                                        w_ref[dy, dx],
                                        preferred_element_type=jnp.float32)
        y = acc * scale_ref[...] + shift_ref[...]
        y = jnp.where(y > 0, y, SLOPE * y)
        if pad_out:
            col = lax.broadcasted_iota(jnp.int32, (n_rows, c_out), 0) % pitch
            y = jnp.where(col < w, y, 0.0)
            o_ref[0, :pitch + 1, :] = jnp.zeros((pitch + 1, c_out), o_ref.dtype)
            o_ref[0, pitch + 1:pitch + 1 + n_rows, :] = y.astype(o_ref.dtype)
            o_ref[0, pitch + 1 + n_rows:, :] = jnp.zeros(
                (o_ref.shape[1] - pitch - 1 - n_rows, c_out), o_ref.dtype)
        else:
            o_ref[0] = y.astype(o_ref.dtype)

    return body


def _fold_bn(conv_bias, gamma, beta, mean, var):
    scale = gamma / jnp.sqrt(var + BN_EPS)
    shift = (conv_bias - mean) * scale + beta
    return (scale.astype(jnp.float32).reshape(1, -1),
            shift.astype(jnp.float32).reshape(1, -1))


def _pad_rows(x_nhwc):
    b, h, w, c = x_nhwc.shape
    xp = jnp.pad(x_nhwc, ((0, 0), (1, 1), (1, 1), (0, 0)))
    xf = xp.reshape(b, (h + 2) * (w + 2), c)
    return jnp.pad(xf, ((0, 0), (0, 2), (0, 0)))


def kernel(x1, x2, w_1x1, b_1x1, w_conv_a, b_conv_a,
           bn_a_gamma, bn_a_beta, bn_a_mean, bn_a_var,
           w_conv_b, b_conv_b, bn_b_gamma, bn_b_beta, bn_b_mean, bn_b_var):
    b, c1, h, w = x1.shape
    c2 = w_1x1.shape[0]
    c_out = w_conv_a.shape[0]
    h2, w2 = 2 * h, 2 * w
    pitch = w2 + 2
    n_pad = (h2 + 2) * (w2 + 2) + 2
    n_rows = h2 * pitch
    par = pltpu.CompilerParams(dimension_semantics=("parallel",))

    w1 = w_1x1[:, :, 0, 0].astype(jnp.bfloat16)
    y1 = pl.pallas_call(
        _conv1x1_kernel,
        out_shape=jax.ShapeDtypeStruct((b, c2, h * w), jnp.float32),
        grid=(b,),
        in_specs=[
            pl.BlockSpec((1, c1, h * w), lambda i: (i, 0, 0)),
            pl.BlockSpec((c2, c1), lambda i: (0, 0)),
            pl.BlockSpec((c2, 1), lambda i: (0, 0)),
        ],
        out_specs=pl.BlockSpec((1, c2, h * w), lambda i: (i, 0, 0)),
        compiler_params=par,
    )(x1.reshape(b, c1, h * w), w1, b_1x1.reshape(c2, 1).astype(jnp.float32))

    uy = _interp_matrix(h)
    uxt = _interp_matrix(w).T
    uyb = jnp.broadcast_to(uy[None], (c2, h2, h))
    y_up = pl.pallas_call(
        _make_upsample_kernel(c2, h, w),
        out_shape=jax.ShapeDtypeStruct((b, c2, h2, w2), jnp.bfloat16),
        grid=(b,),
        in_specs=[
            pl.BlockSpec((1, c2 * h, w), lambda i: (i, 0, 0)),
            pl.BlockSpec((c2, h2, h), lambda i: (0, 0, 0)),
            pl.BlockSpec((w, w2), lambda i: (0, 0)),
        ],
        out_specs=pl.BlockSpec((1, c2, h2, w2), lambda i: (i, 0, 0, 0)),
        compiler_params=par,
    )(y1.reshape(b, c2 * h, w), uyb, uxt)

    x2p = _pad_rows(jnp.transpose(x2.astype(jnp.bfloat16), (0, 2, 3, 1)))
    yupp = _pad_rows(jnp.transpose(y_up, (0, 2, 3, 1)))

    w_a = jnp.transpose(w_conv_a, (2, 3, 1, 0)).astype(jnp.bfloat16)
    scale_a, shift_a = _fold_bn(b_conv_a, bn_a_gamma, bn_a_beta,
                                bn_a_mean, bn_a_var)

    za = pl.pallas_call(
        _make_conv_kernel(2, h2, w2, pad_out=True),
        out_shape=jax.ShapeDtypeStruct((b, n_pad, c_out), jnp.bfloat16),
        grid=(b,),
        in_specs=[
            pl.BlockSpec((1, n_pad, c2), lambda i: (i, 0, 0)),
            pl.BlockSpec((1, n_pad, c2), lambda i: (i, 0, 0)),
            pl.BlockSpec((3, 3, c2, c_out), lambda i: (0, 0, 0, 0)),
            pl.BlockSpec((3, 3, c2, c_out), lambda i: (0, 0, 0, 0)),
            pl.BlockSpec((1, c_out), lambda i: (0, 0)),
            pl.BlockSpec((1, c_out), lambda i: (0, 0)),
        ],
        out_specs=pl.BlockSpec((1, n_pad, c_out), lambda i: (i, 0, 0)),
        compiler_params=par,
    )(x2p, yupp, w_a[:, :, :c2, :], w_a[:, :, c2:, :], scale_a, shift_a)

    w_b = jnp.transpose(w_conv_b, (2, 3, 1, 0)).astype(jnp.bfloat16)
    scale_b, shift_b = _fold_bn(b_conv_b, bn_b_gamma, bn_b_beta,
                                bn_b_mean, bn_b_var)
    zb = pl.pallas_call(
        _make_conv_kernel(1, h2, w2, pad_out=False),
        out_shape=jax.ShapeDtypeStruct((b, n_rows, c_out), jnp.float32),
        grid=(b,),
        in_specs=[
            pl.BlockSpec((1, n_pad, c_out), lambda i: (i, 0, 0)),
            pl.BlockSpec((3, 3, c_out, c_out), lambda i: (0, 0, 0, 0)),
            pl.BlockSpec((1, c_out), lambda i: (0, 0)),
            pl.BlockSpec((1, c_out), lambda i: (0, 0)),
        ],
        out_specs=pl.BlockSpec((1, n_rows, c_out), lambda i: (i, 0, 0)),
        compiler_params=par,
    )(za, w_b, scale_b, shift_b)

    z = zb.reshape(b, h2, pitch, c_out)[:, :, :w2, :]
    return jnp.transpose(z, (0, 3, 1, 2))


# speedup vs baseline: 1.4507x; 1.1325x over previous
import jax
import jax.numpy as jnp
from jax import lax
from jax.experimental import pallas as pl
from jax.experimental.pallas import tpu as pltpu

BN_EPS = 1e-5
SLOPE = 0.01


def _interp_matrix(n_in):
    n_out = 2 * n_in
    s = jnp.arange(n_out, dtype=jnp.float32) * (n_in - 1) / (n_out - 1)
    i0 = jnp.floor(s).astype(jnp.int32)
    i1 = jnp.minimum(i0 + 1, n_in - 1)
    frac = s - i0.astype(jnp.float32)
    m = jnp.zeros((n_out, n_in), jnp.float32)
    m = m.at[jnp.arange(n_out), i0].add(1.0 - frac)
    m = m.at[jnp.arange(n_out), i1].add(frac)
    return m


def _conv1x1_kernel(x_ref, w_ref, b_ref, o_ref):
    x = x_ref[0].astype(jnp.bfloat16)
    y = jnp.dot(w_ref[...], x, preferred_element_type=jnp.float32)
    o_ref[0] = y + b_ref[...]


def _make_upsample_kernel(c2, h, w):
    h2, w2 = 2 * h, 2 * w

    def body(x_ref, uyb_ref, uxt_ref, o_ref):
        t = jnp.dot(x_ref[0], uxt_ref[...],
                    preferred_element_type=jnp.float32)
        t = t.reshape(c2, h, w2)
        o = lax.dot_general(uyb_ref[...], t, (((2,), (1,)), ((0,), (0,))),
                            preferred_element_type=jnp.float32)
        o_ref[0] = o.astype(jnp.bfloat16)

    return body


def _make_conv3x3_kernel(n_in, h, w, cm_in, nchw_out):
    n = h * w
    guard = w + 1

    def body(*refs):
        x_refs = refs[:n_in]
        w_refs = refs[n_in:2 * n_in]
        scale_ref, shift_ref, o_ref = refs[2 * n_in:2 * n_in + 3]
        scratch_refs = refs[2 * n_in + 3:]
        c_out = w_refs[0].shape[-1]

        for x_ref, s_ref in zip(x_refs, scratch_refs):
            xin = x_ref[0]
            if xin.dtype != jnp.bfloat16:
                xin = xin.astype(jnp.bfloat16)
            rows = xin.T if cm_in else xin
            s_ref[:guard, :] = jnp.zeros((guard, rows.shape[1]), jnp.bfloat16)
            s_ref[guard:guard + n, :] = rows
            s_ref[guard + n:, :] = jnp.zeros((guard, rows.shape[1]),
                                             jnp.bfloat16)

        col = lax.broadcasted_iota(jnp.int32, (n, c_out), 0) % w
        acc = jnp.zeros((n, c_out), jnp.float32)
        for dx in range(3):
            part = jnp.zeros((n, c_out), jnp.float32)
            for s_ref, w_ref in zip(scratch_refs, w_refs):
                for dy in range(3):
                    base = guard + (dy - 1) * w + (dx - 1)
                    part = part + jnp.dot(s_ref[base:base + n, :],
                                          w_ref[dy, dx],
                                          preferred_element_type=jnp.float32)
            if dx == 0:
                part = jnp.where(col >= 1, part, 0.0)
            elif dx == 2:
                part = jnp.where(col < w - 1, part, 0.0)
            acc = acc + part

        y = acc * scale_ref[...] + shift_ref[...]
        y = jnp.where(y > 0, y, SLOPE * y)
        if nchw_out:
            o_ref[0] = y.T.astype(o_ref.dtype)
        else:
            o_ref[0] = y.astype(o_ref.dtype)

    return body


def _fold_bn(conv_bias, gamma, beta, mean, var):
    scale = gamma / jnp.sqrt(var + BN_EPS)
    shift = (conv_bias - mean) * scale + beta
    return (scale.astype(jnp.float32).reshape(1, -1),
            shift.astype(jnp.float32).reshape(1, -1))


def kernel(x1, x2, w_1x1, b_1x1, w_conv_a, b_conv_a,
           bn_a_gamma, bn_a_beta, bn_a_mean, bn_a_var,
           w_conv_b, b_conv_b, bn_b_gamma, bn_b_beta, bn_b_mean, bn_b_var):
    b, c1, h, w = x1.shape
    c2 = w_1x1.shape[0]
    c_out = w_conv_a.shape[0]
    h2, w2 = 2 * h, 2 * w
    n = h2 * w2
    guard = w2 + 1
    par = pltpu.CompilerParams(dimension_semantics=("parallel",))

    w1 = w_1x1[:, :, 0, 0].astype(jnp.bfloat16)
    y1 = pl.pallas_call(
        _conv1x1_kernel,
        out_shape=jax.ShapeDtypeStruct((b, c2, h * w), jnp.float32),
        grid=(b,),
        in_specs=[
            pl.BlockSpec((1, c1, h * w), lambda i: (i, 0, 0)),
            pl.BlockSpec((c2, c1), lambda i: (0, 0)),
            pl.BlockSpec((c2, 1), lambda i: (0, 0)),
        ],
        out_specs=pl.BlockSpec((1, c2, h * w), lambda i: (i, 0, 0)),
        compiler_params=par,
    )(x1.reshape(b, c1, h * w), w1, b_1x1.reshape(c2, 1).astype(jnp.float32))

    uy = _interp_matrix(h)
    uxt = _interp_matrix(w).T
    uyb = jnp.broadcast_to(uy[None], (c2, h2, h))
    y_up = pl.pallas_call(
        _make_upsample_kernel(c2, h, w),
        out_shape=jax.ShapeDtypeStruct((b, c2, h2, w2), jnp.bfloat16),
        grid=(b,),
        in_specs=[
            pl.BlockSpec((1, c2 * h, w), lambda i: (i, 0, 0)),
            pl.BlockSpec((c2, h2, h), lambda i: (0, 0, 0)),
            pl.BlockSpec((w, w2), lambda i: (0, 0)),
        ],
        out_specs=pl.BlockSpec((1, c2, h2, w2), lambda i: (i, 0, 0, 0)),
        compiler_params=par,
    )(y1.reshape(b, c2 * h, w), uyb, uxt)

    w_a = jnp.transpose(w_conv_a, (2, 3, 1, 0)).astype(jnp.bfloat16)
    scale_a, shift_a = _fold_bn(b_conv_a, bn_a_gamma, bn_a_beta,
                                bn_a_mean, bn_a_var)
    za = pl.pallas_call(
        _make_conv3x3_kernel(2, h2, w2, cm_in=True, nchw_out=False),
        out_shape=jax.ShapeDtypeStruct((b, n, c_out), jnp.bfloat16),
        grid=(b,),
        in_specs=[
            pl.BlockSpec((1, c2, n), lambda i: (i, 0, 0)),
            pl.BlockSpec((1, c2, n), lambda i: (i, 0, 0)),
            pl.BlockSpec((3, 3, c2, c_out), lambda i: (0, 0, 0, 0)),
            pl.BlockSpec((3, 3, c2, c_out), lambda i: (0, 0, 0, 0)),
            pl.BlockSpec((1, c_out), lambda i: (0, 0)),
            pl.BlockSpec((1, c_out), lambda i: (0, 0)),
        ],
        out_specs=pl.BlockSpec((1, n, c_out), lambda i: (i, 0, 0)),
        compiler_params=par,
        scratch_shapes=[pltpu.VMEM((n + 2 * guard, c2), jnp.bfloat16),
                        pltpu.VMEM((n + 2 * guard, c2), jnp.bfloat16)],
    )(x2.reshape(b, c2, n), y_up.reshape(b, c2, n),
      w_a[:, :, :c2, :], w_a[:, :, c2:, :], scale_a, shift_a)

    w_b = jnp.transpose(w_conv_b, (2, 3, 1, 0)).astype(jnp.bfloat16)
    scale_b, shift_b = _fold_bn(b_conv_b, bn_b_gamma, bn_b_beta,
                                bn_b_mean, bn_b_var)
    zb = pl.pallas_call(
        _make_conv3x3_kernel(1, h2, w2, cm_in=False, nchw_out=True),
        out_shape=jax.ShapeDtypeStruct((b, c_out, n), jnp.float32),
        grid=(b,),
        in_specs=[
            pl.BlockSpec((1, n, c_out), lambda i: (i, 0, 0)),
            pl.BlockSpec((3, 3, c_out, c_out), lambda i: (0, 0, 0, 0)),
            pl.BlockSpec((1, c_out), lambda i: (0, 0)),
            pl.BlockSpec((1, c_out), lambda i: (0, 0)),
        ],
        out_specs=pl.BlockSpec((1, c_out, n), lambda i: (i, 0, 0)),
        compiler_params=par,
        scratch_shapes=[pltpu.VMEM((n + 2 * guard, c_out), jnp.bfloat16)],
    )(za, w_b, scale_b, shift_b)

    return zb.reshape(b, c_out, h2, w2)
